# Initial kernel scaffold; baseline (speedup 1.0000x reference)
import jax
import jax.numpy as jnp
from jax.experimental import pallas as pl


def kernel(x, edge_index, W0a, g0m, b0m, W0b, g0, b0, W1a, g1m, b1m, W1b, g1, b1):
    raise NotImplementedError("write your pallas kernel here")


# speedup vs baseline: 3.7643x; 3.7643x over previous
import functools

import jax
import jax.numpy as jnp
from jax import lax
from jax.experimental import pallas as pl
from jax.experimental.pallas import tpu as pltpu
from jax.experimental.pallas import tpu_sc as plsc


def _row_split(n, ns):
    assert n % 8 == 0
    ngrp = n // 8
    return (ngrp // ns) * 8, ngrp % ns


def _chunk_size(epw):
    return max(c for c in range(8, 129, 8) if epw % c == 0)


@functools.lru_cache(maxsize=None)
def _make_sc_agg_featsplit(n, e, d2):
    info = plsc.get_sparse_core_info()
    ns = info.num_subcores
    assert e % ns == 0 and d2 % 128 == 0
    epw = e // ns
    k = _chunk_size(epw)
    nchunk = epw // k
    rpw, grem = _row_split(n, ns)

    mesh = plsc.VectorSubcoreMesh(core_axis_name="c", subcore_axis_name="s")

    def body(t0, t1, src_hbm, dst_hbm, out0, out1, acc, idx_s, idx_d, rows, sem):
        cid = lax.axis_index("c")
        sid = lax.axis_index("s")

        def work(tab, out):
            r0 = sid * rpw
            rem0 = ns * rpw + sid * 8
            pltpu.sync_copy(tab.at[pl.ds(r0, rpw)], acc.at[pl.ds(r0, rpw)])
            if grem:
                @pl.when(sid < grem)
                def _():
                    pltpu.sync_copy(tab.at[pl.ds(rem0, 8)],
                                    acc.at[pl.ds(rem0, 8)])
            plsc.subcore_barrier()

            def chunk(i, carry):
                base = sid * epw + i * k
                pltpu.sync_copy(src_hbm.at[pl.ds(base, k)], idx_s)
                pltpu.sync_copy(dst_hbm.at[pl.ds(base, k)], idx_d)
                pltpu.async_copy(tab.at[idx_s], rows, sem).wait()
                pltpu.sync_copy(rows, acc.at[idx_d], add=True)
                return carry

            lax.fori_loop(0, nchunk, chunk, 0)
            plsc.subcore_barrier()
            pltpu.sync_copy(acc.at[pl.ds(r0, rpw)], out.at[pl.ds(r0, rpw)])
            if grem:
                @pl.when(sid < grem)
                def _():
                    pltpu.sync_copy(acc.at[pl.ds(rem0, 8)],
                                    out.at[pl.ds(rem0, 8)])

        @pl.when(cid == 0)
        def _():
            work(t0, out0)

        @pl.when(cid == 1)
        def _():
            work(t1, out1)

    return pl.kernel(
        body,
        out_type=(jax.ShapeDtypeStruct((n, d2), jnp.float32),
                  jax.ShapeDtypeStruct((n, d2), jnp.float32)),
        mesh=mesh,
        scratch_types=[
            pltpu.VMEM_SHARED((n, d2), jnp.float32),
            pltpu.VMEM((k,), jnp.int32),
            pltpu.VMEM((k,), jnp.int32),
            pltpu.VMEM((k, d2), jnp.float32),
            pltpu.SemaphoreType.DMA,
        ],
    )


@functools.lru_cache(maxsize=None)
def _make_sc_agg_edgesplit(n, e, d):
    info = plsc.get_sparse_core_info()
    nc, ns = info.num_cores, info.num_subcores
    nw = nc * ns
    assert e % nw == 0 and d % 128 == 0
    epw = e // nw
    k = _chunk_size(epw)
    nchunk = epw // k
    rpw, grem = _row_split(n, ns)

    mesh = plsc.VectorSubcoreMesh(core_axis_name="c", subcore_axis_name="s")

    def body(tab, zer, src_hbm, dst_hbm, out0, out1, acc, idx_s, idx_d, rows,
             sem):
        cid = lax.axis_index("c")
        sid = lax.axis_index("s")
        r0 = sid * rpw
        rem0 = ns * rpw + sid * 8

        def init(seed_ref):
            pltpu.sync_copy(seed_ref.at[pl.ds(r0, rpw)], acc.at[pl.ds(r0, rpw)])
            if grem:
                @pl.when(sid < grem)
                def _():
                    pltpu.sync_copy(seed_ref.at[pl.ds(rem0, 8)],
                                    acc.at[pl.ds(rem0, 8)])

        @pl.when(cid == 0)
        def _():
            init(tab)

        @pl.when(cid == 1)
        def _():
            init(zer)

        plsc.subcore_barrier()

        wid = cid * ns + sid

        def chunk(i, carry):
            base = wid * epw + i * k
            pltpu.sync_copy(src_hbm.at[pl.ds(base, k)], idx_s)
            pltpu.sync_copy(dst_hbm.at[pl.ds(base, k)], idx_d)
            pltpu.async_copy(tab.at[idx_s], rows, sem).wait()
            pltpu.sync_copy(rows, acc.at[idx_d], add=True)
            return carry

        lax.fori_loop(0, nchunk, chunk, 0)
        plsc.subcore_barrier()

        def copyout(out):
            pltpu.sync_copy(acc.at[pl.ds(r0, rpw)], out.at[pl.ds(r0, rpw)])
            if grem:
                @pl.when(sid < grem)
                def _():
                    pltpu.sync_copy(acc.at[pl.ds(rem0, 8)],
                                    out.at[pl.ds(rem0, 8)])

        @pl.when(cid == 0)
        def _():
            copyout(out0)

        @pl.when(cid == 1)
        def _():
            copyout(out1)

    return pl.kernel(
        body,
        out_type=(jax.ShapeDtypeStruct((n, d), jnp.float32),
                  jax.ShapeDtypeStruct((n, d), jnp.float32)),
        mesh=mesh,
        scratch_types=[
            pltpu.VMEM_SHARED((n, d), jnp.float32),
            pltpu.VMEM((k,), jnp.int32),
            pltpu.VMEM((k,), jnp.int32),
            pltpu.VMEM((k, d), jnp.float32),
            pltpu.SemaphoreType.DMA,
        ],
    )


def _bn_relu(z, g, b):
    mu = jnp.mean(z, axis=0, keepdims=True)
    var = jnp.mean((z - mu) ** 2, axis=0, keepdims=True)
    return jnp.maximum((z - mu) * lax.rsqrt(var + 1e-5) * g + b, 0.0)


@functools.lru_cache(maxsize=None)
def _make_mlp(n, d_in2, d_h, d_out, combine, split_out):
    def body(a0, a1, wa, wb, gm, bm, g, b, *outs):
        if combine == "concat":
            s = jnp.concatenate([a0[...], a1[...]], axis=1)
        else:
            s = a0[...] + a1[...]
        z = jnp.dot(s, wa[...], preferred_element_type=jnp.float32)
        z = _bn_relu(z, gm[...], bm[...])
        h = jnp.dot(z, wb[...], preferred_element_type=jnp.float32)
        h = _bn_relu(h, g[...], b[...])
        if split_out:
            outs[0][...] = h[:, : d_out // 2]
            outs[1][...] = h[:, d_out // 2:]
        else:
            outs[0][...] = h

    if split_out:
        out_shape = (jax.ShapeDtypeStruct((n, d_out // 2), jnp.float32),
                     jax.ShapeDtypeStruct((n, d_out // 2), jnp.float32))
    else:
        out_shape = jax.ShapeDtypeStruct((n, d_out), jnp.float32)
    return pl.pallas_call(body, out_shape=out_shape)


def kernel(x, edge_index, W0a, g0m, b0m, W0b, g0, b0, W1a, g1m, b1m, W1b, g1, b1):
    n, d_in = x.shape
    e = edge_index.shape[1]
    d_h = W0a.shape[1]
    d_out = W1b.shape[1]
    src = edge_index[0]
    dst = edge_index[1]

    r2 = lambda v: v.reshape(1, -1)

    zer = jnp.zeros_like(x)
    a0, a1 = _make_sc_agg_edgesplit(n, e, d_in)(x, zer, src, dst)
    h0, h1 = _make_mlp(n, d_in, d_h, d_h, "add", True)(
        a0, a1, W0a, W0b, r2(g0m), r2(b0m), r2(g0), r2(b0))

    b0_, b1_ = _make_sc_agg_featsplit(n, e, d_h // 2)(h0, h1, src, dst)
    out = _make_mlp(n, d_h // 2, d_h, d_out, "concat", False)(
        b0_, b1_, W1a, W1b, r2(g1m), r2(b1m), r2(g1), r2(b1))
    return out
